# baseline (speedup 1.0000x reference)
import jax
import jax.numpy as jnp
from jax import lax
from jax.experimental import pallas as pl
from jax.experimental.pallas import tpu as pltpu
from jax.experimental.pallas import tpu_sc as plsc

L = 512
LL = L * L
NB_THETA, NB_PHI, NB_DIST, NB_OMEGA = 25, 13, 37, 25
NC, NS, LANES = 2, 16, 16
NW = NC * NS
P = LL // NW
NVEC = P // LANES
CHUNK = 128
NCHUNK = P // CHUNK

_C5 = (0.04392863, -0.40947559, 1.61017755, -3.52021884, 5.06975632,
       -2.79415368)
_LN2 = 0.6931471805599453


def _softlog(x):
    xi = lax.bitcast_convert_type(x, jnp.int32)
    e = ((xi >> 23) - 127).astype(jnp.float32)
    m = lax.bitcast_convert_type((xi & 0x007FFFFF) | 0x3F800000, jnp.float32)
    p = jnp.float32(_C5[0])
    for c in _C5[1:]:
        p = p * m + jnp.float32(c)
    return jnp.float32(_LN2) * (e + p)


def _sc_body(t_tab, p_tab, d_tab, o_tab, mask_hbm, t_idx, p_idx, d_idx,
             o_idx, out_hbm, mask_v, idx_v, fidx, gbuf, out_v, sem):
    wid = lax.axis_index("s") * NC + lax.axis_index("c")
    base = wid * P
    pltpu.sync_copy(mask_hbm.at[pl.ds(base, P)], mask_v)
    iota = lax.iota(jnp.int32, LANES)
    total = jnp.zeros((LANES,), jnp.float32)

    for tab, idxh in ((t_tab, t_idx), (p_tab, p_idx),
                      (d_tab, d_idx), (o_tab, o_idx)):
        pltpu.sync_copy(idxh.at[pl.ds(base, P)], idx_v)

        def flat_body(i, c):
            v = idx_v[pl.ds(i * LANES, LANES)]
            fidx[pl.ds(i * LANES, LANES)] = v * LL + (base + i * LANES) + iota
            return c

        lax.fori_loop(0, NVEC, flat_body, jnp.int32(0))

        def fire_body(j, c):
            pltpu.async_copy(tab.at[fidx.at[pl.ds(j * CHUNK, CHUNK)]],
                             gbuf.at[pl.ds(j * CHUNK, CHUNK)], sem)
            return c

        lax.fori_loop(0, NCHUNK, fire_body, jnp.int32(0))

        def drain_body(j, c):
            pltpu.make_async_copy(tab.at[fidx.at[pl.ds(j * CHUNK, CHUNK)]],
                                  gbuf.at[pl.ds(j * CHUNK, CHUNK)], sem).wait()
            return c

        lax.fori_loop(0, NCHUNK, drain_body, jnp.int32(0))

        def log_body(i, s):
            g = gbuf[pl.ds(i * LANES, LANES)]
            m = mask_v[pl.ds(i * LANES, LANES)]
            return s + m * _softlog(g)

        total = total + lax.fori_loop(0, NVEC, log_body,
                                      jnp.zeros((LANES,), jnp.float32))

    out_v[...] = total
    pltpu.sync_copy(out_v, out_hbm.at[pl.ds(wid * LANES, LANES)])


@jax.jit
def kernel(theta, phi, dist, omega, mask, idx_theta, idx_phi, idx_dist, idx_omega):
    mesh = plsc.VectorSubcoreMesh(core_axis_name="c", subcore_axis_name="s",
                                  num_cores=NC, num_subcores=NS)
    run = pl.kernel(
        _sc_body, mesh=mesh,
        out_type=jax.ShapeDtypeStruct((NW * LANES,), jnp.float32),
        scratch_types=[
            pltpu.VMEM((P,), jnp.float32),
            pltpu.VMEM((P,), jnp.int32),
            pltpu.VMEM((P,), jnp.int32),
            pltpu.VMEM((P,), jnp.float32),
            pltpu.VMEM((LANES,), jnp.float32),
            pltpu.SemaphoreType.DMA,
        ],
    )
    out = run(
        theta.reshape(NB_THETA * LL),
        phi.reshape(NB_PHI * LL),
        dist.reshape(NB_DIST * LL),
        omega.reshape(NB_OMEGA * LL),
        mask.reshape(LL),
        idx_theta.reshape(LL).astype(jnp.int32),
        idx_phi.reshape(LL).astype(jnp.int32),
        idx_dist.reshape(LL).astype(jnp.int32),
        idx_omega.reshape(LL).astype(jnp.int32),
    )
    return -jnp.sum(out) / jnp.float32(LL)
